# baseline (speedup 1.0000x reference)
import functools

import jax
import jax.numpy as jnp
from jax import lax
from jax.experimental import pallas as pl
from jax.experimental.pallas import tpu as pltpu
from jax.experimental.pallas import tpu_sc as plsc

_INTERPRET = False

L = 16
NT = 16
NEG = -1e30


def _tc_a(x, W0, W1s, b1s, W1d, b1d):
    N, D_IN = x.shape
    D = W0.shape[1]
    BLK = 1000

    def body(x_ref, w0_ref, w1s_ref, b1s_ref, w1d_ref, b1d_ref, hs_ref, hd_ref):
        h = jnp.dot(x_ref[...], w0_ref[...], preferred_element_type=jnp.float32)
        hs_ref[...] = jnp.dot(h, w1s_ref[...], preferred_element_type=jnp.float32) + b1s_ref[...]
        hd_ref[...] = jnp.dot(h, w1d_ref[...], preferred_element_type=jnp.float32) + b1d_ref[...]

    return pl.pallas_call(
        body,
        grid=(N // BLK,),
        in_specs=[
            pl.BlockSpec((BLK, D_IN), lambda i: (i, 0)),
            pl.BlockSpec((D_IN, D), lambda i: (0, 0)),
            pl.BlockSpec((D, D), lambda i: (0, 0)),
            pl.BlockSpec((1, D), lambda i: (0, 0)),
            pl.BlockSpec((D, D), lambda i: (0, 0)),
            pl.BlockSpec((1, D), lambda i: (0, 0)),
        ],
        out_specs=[
            pl.BlockSpec((BLK, D), lambda i: (i, 0)),
            pl.BlockSpec((BLK, D), lambda i: (i, 0)),
        ],
        out_shape=[
            jax.ShapeDtypeStruct((N, D), jnp.float32),
            jax.ShapeDtypeStruct((N, D), jnp.float32),
        ],
        interpret=_INTERPRET,
    )(x, W0, W1s, b1s.reshape(1, D), W1d, b1d.reshape(1, D))


def _tc_b(out1, c1, W2s, b2s, W2d, b2d):
    N, D = out1.shape
    BLK = 1000

    def body(o_ref, c1_ref, w2s_ref, b2s_ref, w2d_ref, b2d_ref, hs_ref, hd_ref):
        h1 = jax.nn.gelu(o_ref[...] + c1_ref[...])
        hs_ref[...] = jnp.dot(h1, w2s_ref[...], preferred_element_type=jnp.float32) + b2s_ref[...]
        hd_ref[...] = jnp.dot(h1, w2d_ref[...], preferred_element_type=jnp.float32) + b2d_ref[...]

    return pl.pallas_call(
        body,
        grid=(N // BLK,),
        in_specs=[
            pl.BlockSpec((BLK, D), lambda i: (i, 0)),
            pl.BlockSpec((1, D), lambda i: (0, 0)),
            pl.BlockSpec((D, D), lambda i: (0, 0)),
            pl.BlockSpec((1, D), lambda i: (0, 0)),
            pl.BlockSpec((D, D), lambda i: (0, 0)),
            pl.BlockSpec((1, D), lambda i: (0, 0)),
        ],
        out_specs=[
            pl.BlockSpec((BLK, D), lambda i: (i, 0)),
            pl.BlockSpec((BLK, D), lambda i: (i, 0)),
        ],
        out_shape=[
            jax.ShapeDtypeStruct((N, D), jnp.float32),
            jax.ShapeDtypeStruct((N, D), jnp.float32),
        ],
        interpret=_INTERPRET,
    )(out1, c1.reshape(1, D), W2s, b2s.reshape(1, D), W2d, b2d.reshape(1, D))


def _lrelu(t):
    return jnp.maximum(t, 0.2 * t)


def _gelu(t):
    u = 0.7978845608028654 * (t + 0.044715 * t * t * t)
    th = 1.0 - 2.0 / (jnp.exp(2.0 * u) + 1.0)
    return 0.5 * t * (1.0 + th)


def _vsum(v):
    s = v[0]
    for k in range(1, L):
        s = s + v[k]
    return s


def _mesh():
    return plsc.VectorSubcoreMesh(core_axis_name="c", subcore_axis_name="s",
                                  num_cores=2, num_subcores=16)


def _make_sc_a(N, E, D):
    UM = 12288
    DUMP = N
    EW = E // NT
    CH = 2000
    SCH = 64
    STG = 2048
    IDS_W = EW + STG
    RPT = (N + L) // NT

    scratch = [
        pltpu.VMEM_SHARED((N + L, D), jnp.float32),
        pltpu.VMEM_SHARED((UM,), jnp.int32),
        pltpu.VMEM_SHARED((UM,), jnp.float32),
        pltpu.VMEM_SHARED((NT * L,), jnp.float32),
        pltpu.HBM((NT * IDS_W,), jnp.float32),
        pltpu.VMEM((CH,), jnp.int32),
        pltpu.VMEM((CH,), jnp.int32),
        pltpu.VMEM((10240,), jnp.int32),
        pltpu.VMEM((STG + L,), jnp.int32),
        pltpu.VMEM((L,), jnp.int32),
        pltpu.VMEM((L,), jnp.int32),
        pltpu.VMEM((SCH,), jnp.int32),
        pltpu.VMEM((SCH,), jnp.int32),
        pltpu.VMEM((SCH,), jnp.int32),
        pltpu.VMEM((SCH,), jnp.int32),
        pltpu.VMEM((SCH,), jnp.float32),
        pltpu.VMEM((SCH, D), jnp.float32),
        pltpu.VMEM((SCH, D), jnp.float32),
        pltpu.VMEM((L, D), jnp.float32),
        pltpu.VMEM((SCH,), jnp.float32),
        pltpu.VMEM((SCH,), jnp.float32),
        pltpu.VMEM((L,), jnp.int32),
        pltpu.VMEM((D,), jnp.float32),
        pltpu.VMEM((L,), jnp.int32),
        pltpu.VMEM((L,), jnp.float32),
        pltpu.VMEM((STG,), jnp.float32),
        pltpu.VMEM((NT * L,), jnp.float32),
        pltpu.VMEM((CH,), jnp.int32),
        pltpu.VMEM((CH // L + 3,), jnp.int32),
        pltpu.VMEM((CH,), jnp.int32),
        pltpu.SemaphoreType.DMA,
        pltpu.SemaphoreType.DMA,
    ]

    @functools.partial(
        pl.kernel,
        out_type=(
            jax.ShapeDtypeStruct((N, D), jnp.float32),
            jax.ShapeDtypeStruct((NT * IDS_W,), jnp.int32),
            jax.ShapeDtypeStruct((NT * L,), jnp.int32),
        ),
        mesh=_mesh(),
        scratch_types=scratch,
        compiler_params=pltpu.CompilerParams(needs_layout_passes=False),
        interpret=_INTERPRET,
    )
    def sc_a(src_hbm, dst_hbm, nidx_hbm, hs1_hbm, hd1_hbm, a1_hbm,
             out1_hbm, ids_hbm, nfo_hbm,
             out1_sh, umask_sh, denom_sh, m1_sh, ebuf_hb,
             dstc, srcc, umask, stage, stg16, onesv,
             idsc, sv, dv, dvw, denc, rows_s, rows_d, rows16,
             echunk, alphac, nidxv, a1c, cntst, m1st, zstagef, m1all, cbuf, cntb,
             dstc2, semA, semB):
        cid = lax.axis_index("c")
        w = lax.axis_index("s")

        def popcnt(m):
            return plsc.all_reduce_population_count(m)[0]

        @pl.when(cid == 0)
        def _main():
            iot = lax.iota(jnp.int32, L)
            zi = jnp.zeros((L,), jnp.int32)
            zf = jnp.zeros((L,), jnp.float32)
            onei = jnp.ones((L,), jnp.int32)
            ebase = w * EW

            pltpu.sync_copy(nidx_hbm, nidxv)
            pltpu.sync_copy(a1_hbm, a1c)
            nvec = nidxv[:]
            onesv[:] = onei

            def stg_body(j, _):
                stage[pl.ds(j * L, L)] = zi
                return 0
            lax.fori_loop(0, (STG + L) // L, stg_body, 0)

            def stgf_body(j, _):
                zstagef[pl.ds(j * L, L)] = zf
                return 0
            lax.fori_loop(0, STG // L, stgf_body, 0)

            def zrow_body(j, _):
                def zcol(k, _2):
                    rows16[j, pl.ds(k * L, L)] = zf
                    return 0
                lax.fori_loop(0, D // L, zcol, 0)
                return 0
            lax.fori_loop(0, L, zrow_body, 0)

            @pl.when(w == 0)
            def _zero_shared():
                def zs(j, _):
                    pltpu.sync_copy(stage.at[pl.ds(0, STG)],
                                    umask_sh.at[pl.ds(j * STG, STG)])
                    return 0
                lax.fori_loop(0, UM // STG, zs, 0)

                def zd(j, _):
                    pltpu.sync_copy(zstagef.at[pl.ds(0, STG)],
                                    denom_sh.at[pl.ds(j * STG, STG)])
                    return 0
                lax.fori_loop(0, UM // STG, zd, 0)

            @pl.when(w == 0)
            def _zero_dump():
                pltpu.sync_copy(rows16, out1_sh.at[pl.ds(N, L)])

            plsc.subcore_barrier()

            fvec = jnp.zeros((L,), jnp.bool_)
            NCH = EW // CH

            def p1_proc(t, dref):
                off = pl.multiple_of(ebase + t * CH, 8)

                def p1_scan(j, macc):
                    return macc | (dref[pl.ds(j * L, L)] == nvec)
                macc = lax.fori_loop(0, CH // L, p1_scan, fvec)

                @pl.when(popcnt(macc) > 0)
                def _rescan():
                    pltpu.sync_copy(src_hbm.at[pl.ds(off, CH)], srcc)

                    def p1_inner(j, _2):
                        d16 = dref[pl.ds(j * L, L)]
                        m = d16 == nvec

                        @pl.when(popcnt(m) > 0)
                        def _mark():
                            s16 = srcc[pl.ds(j * L, L)]
                            idxw = jnp.where(m, s16, DUMP)
                            pltpu.sync_copy(onesv, umask_sh.at[idxw])
                        return 0
                    lax.fori_loop(0, CH // L, p1_inner, 0)

            def p1_start(t, dref, sem):
                off = pl.multiple_of(ebase + (t % NCH) * CH, 8)
                return pltpu.async_copy(dst_hbm.at[pl.ds(off, CH)], dref, sem)

            p1_start(0, dstc, semA)

            def p1_pair(p, _):
                pltpu.make_async_copy(dst_hbm.at[pl.ds(0, CH)], dstc, semA).wait()
                p1_start(2 * p + 1, dstc2, semB)
                p1_proc(2 * p, dstc)
                pltpu.make_async_copy(dst_hbm.at[pl.ds(0, CH)], dstc2, semB).wait()
                p1_start(2 * p + 2, dstc, semA)
                p1_proc(2 * p + 1, dstc2)
                return 0
            lax.fori_loop(0, NCH // 2, p1_pair, 0)
            pltpu.make_async_copy(dst_hbm.at[pl.ds(0, CH)], dstc, semA).wait()

            plsc.subcore_barrier()

            pltpu.sync_copy(umask_sh.at[pl.ds(0, 10240)], umask)
            plsc.store_scatter(umask, [nvec], onei)

            UG = 10240 // (L * NT)
            def zu_body(g, _):
                gg = w * UG + g
                um16 = umask[pl.ds(gg * L, L)]
                m = um16 > 0

                @pl.when(popcnt(m) > 0)
                def _z():
                    idxw = jnp.where(m, gg * L + iot, DUMP)
                    pltpu.sync_copy(rows16, out1_sh.at[idxw])
                return 0
            lax.fori_loop(0, UG, zu_body, 0)
            plsc.subcore_barrier()

            def p2_outer(t, carry, dref):
                def p2_stageA(j, _2):
                    d16 = dref[pl.ds(j * L, L)]
                    um = plsc.load_gather(umask, [d16])
                    m = um > 0
                    eid = ebase + t * CH + j * L + iot
                    plsc.store_compressed(cbuf.at[pl.ds(j * L, L)], eid, mask=m)
                    cs = plsc.all_reduce_population_count(m)
                    plsc.store_scatter(cntb, [jnp.full((L,), j, jnp.int32)],
                                       cs, mask=iot == 0)
                    return 0
                lax.fori_loop(0, CH // L, p2_stageA, 0)

                def p2_stageB(j, carry2):
                    cnt2, off2, fb2 = carry2
                    c = plsc.load_gather(cntb, [jnp.full((L,), j, jnp.int32)])[0]
                    stage[pl.ds(off2, L)] = cbuf[pl.ds(j * L, L)]
                    off3 = off2 + c

                    @pl.when(off3 >= STG)
                    def _flush():
                        pltpu.sync_copy(
                            stage.at[pl.ds(0, STG)],
                            ids_hbm.at[pl.ds(pl.multiple_of(w * IDS_W + fb2, 8), STG)])
                        stage[pl.ds(0, L)] = stage[pl.ds(STG, L)]

                    wrapped = off3 >= STG
                    off4 = jnp.where(wrapped, off3 - STG, off3)
                    fb3 = jnp.where(wrapped, fb2 + STG, fb2)
                    return (cnt2 + c, off4, fb3)
                return lax.fori_loop(0, CH // L, p2_stageB, carry)

            p1_start(0, dstc, semA)

            def p2_pair(p, carry):
                pltpu.make_async_copy(dst_hbm.at[pl.ds(0, CH)], dstc, semA).wait()
                p1_start(2 * p + 1, dstc2, semB)
                carry = p2_outer(2 * p, carry, dstc)
                pltpu.make_async_copy(dst_hbm.at[pl.ds(0, CH)], dstc2, semB).wait()
                p1_start(2 * p + 2, dstc, semA)
                carry = p2_outer(2 * p + 1, carry, dstc2)
                return carry
            n1, offr, fbr = lax.fori_loop(0, NCH // 2, p2_pair,
                                          (jnp.int32(0), jnp.int32(0), jnp.int32(0)))
            pltpu.make_async_copy(dst_hbm.at[pl.ds(0, CH)], dstc, semA).wait()
            pltpu.sync_copy(stage.at[pl.ds(0, STG)],
                            ids_hbm.at[pl.ds(pl.multiple_of(w * IDS_W + fbr, 8), STG)])

            cntst[:] = jnp.full((L,), n1, jnp.int32)

            ntrip = (n1 + SCH - 1) // SCH

            def p3_outer(t, m1v):
                base = pl.multiple_of(t * SCH, 8)
                pltpu.sync_copy(ids_hbm.at[pl.ds(pl.multiple_of(w * IDS_W + base, 8), SCH)], idsc)

                def _cl(g, _0):
                    v = idsc[pl.ds(g * L, L)]
                    idsc[pl.ds(g * L, L)] = jnp.clip(v, 0, E - 1)
                    return 0
                lax.fori_loop(0, SCH // L, _cl, 0)
                pltpu.sync_copy(src_hbm.at[idsc], sv)
                pltpu.sync_copy(dst_hbm.at[idsc], dv)
                pltpu.sync_copy(hs1_hbm.at[sv], rows_s)
                pltpu.sync_copy(hd1_hbm.at[dv], rows_d)

                def p3_edge(i, m1i):
                    acc = zf
                    for jj in range(D // L):
                        t1 = rows_s[i, pl.ds(jj * L, L)] + rows_d[i, pl.ds(jj * L, L)]
                        acc = acc + _lrelu(t1) * a1c[pl.ds(jj * L, L)]
                    e = _vsum(acc)
                    plsc.store_scatter(echunk, [jnp.full((L,), i, jnp.int32)],
                                       jnp.full((L,), e, jnp.float32), mask=iot == 0)
                    valid = (base + i) < n1
                    ev = jnp.full((L,), jnp.where(valid, e, NEG), jnp.float32)
                    return jnp.maximum(m1i, ev)
                m1v = lax.fori_loop(0, SCH, p3_edge, m1v)
                pltpu.sync_copy(echunk, ebuf_hb.at[pl.ds(pl.multiple_of(w * IDS_W + base, 8), SCH)])
                return m1v
            m1v = lax.fori_loop(0, ntrip, p3_outer,
                                jnp.full((L,), NEG, jnp.float32))
            m1st[:] = m1v
            pltpu.sync_copy(m1st, m1_sh.at[pl.ds(pl.multiple_of(w * L, 8), L)])
            plsc.subcore_barrier()
            pltpu.sync_copy(m1_sh, m1all)
            m1g = jnp.full((L,), NEG, jnp.float32)
            for r in range(NT):
                m1g = jnp.maximum(m1g, m1all[pl.ds(r * L, L)])

            def p4_outer(t, _):
                base = pl.multiple_of(t * SCH, 8)
                pltpu.sync_copy(ids_hbm.at[pl.ds(pl.multiple_of(w * IDS_W + base, 8), SCH)], idsc)

                def _cl(g, _0):
                    v = idsc[pl.ds(g * L, L)]
                    idsc[pl.ds(g * L, L)] = jnp.clip(v, 0, E - 1)
                    return 0
                lax.fori_loop(0, SCH // L, _cl, 0)
                pltpu.sync_copy(dst_hbm.at[idsc], dv)
                pltpu.sync_copy(ebuf_hb.at[pl.ds(pl.multiple_of(w * IDS_W + base, 8), SCH)], echunk)

                def p4_grp(j, _2):
                    e16 = echunk[pl.ds(j * L, L)]
                    d16 = dv[pl.ds(j * L, L)]
                    vmask = (base + j * L + iot) < n1
                    p = jnp.where(vmask, jnp.exp(e16 - m1g), 0.0)
                    echunk[pl.ds(j * L, L)] = p
                    dvw[pl.ds(j * L, L)] = jnp.where(vmask, d16, DUMP)
                    return 0
                lax.fori_loop(0, SCH // L, p4_grp, 0)
                pltpu.sync_copy(echunk, ebuf_hb.at[pl.ds(pl.multiple_of(w * IDS_W + base, 8), SCH)])
                pltpu.sync_copy(echunk, denom_sh.at[dvw], add=True)
                return 0
            lax.fori_loop(0, ntrip, p4_outer, 0)
            plsc.subcore_barrier()

            def p5_outer(t, _):
                base = pl.multiple_of(t * SCH, 8)
                pltpu.sync_copy(ids_hbm.at[pl.ds(pl.multiple_of(w * IDS_W + base, 8), SCH)], idsc)

                def _cl(g, _0):
                    v = idsc[pl.ds(g * L, L)]
                    idsc[pl.ds(g * L, L)] = jnp.clip(v, 0, E - 1)
                    return 0
                lax.fori_loop(0, SCH // L, _cl, 0)
                pltpu.sync_copy(src_hbm.at[idsc], sv)
                pltpu.sync_copy(dst_hbm.at[idsc], dv)
                pltpu.sync_copy(ebuf_hb.at[pl.ds(pl.multiple_of(w * IDS_W + base, 8), SCH)], echunk)
                pltpu.sync_copy(hs1_hbm.at[sv], rows_s)

                def p5_grp(j, _2):
                    d16 = dv[pl.ds(j * L, L)]
                    vmask = (base + j * L + iot) < n1
                    dvw[pl.ds(j * L, L)] = jnp.where(vmask, d16, DUMP)
                    return 0
                lax.fori_loop(0, SCH // L, p5_grp, 0)
                pltpu.sync_copy(denom_sh.at[dvw], denc)

                def p5_alpha(j, _2):
                    p16 = echunk[pl.ds(j * L, L)]
                    den = denc[pl.ds(j * L, L)]
                    vmask = (base + j * L + iot) < n1
                    al = jnp.where(vmask, p16 / (den + 1e-9), 0.0)
                    alphac[pl.ds(j * L, L)] = al
                    return 0
                lax.fori_loop(0, SCH // L, p5_alpha, 0)

                def p5_edge(i, _2):
                    asp = plsc.load_gather(alphac, [jnp.full((L,), i, jnp.int32)])
                    for jj in range(D // L):
                        rows_d[i, pl.ds(jj * L, L)] = rows_s[i, pl.ds(jj * L, L)] * asp
                    return 0
                lax.fori_loop(0, SCH, p5_edge, 0)
                pltpu.sync_copy(rows_d, out1_sh.at[dvw], add=True)
                return 0
            lax.fori_loop(0, ntrip, p5_outer, 0)
            plsc.subcore_barrier()

            NR = (N // NT) // 8 * 8
            pltpu.sync_copy(out1_sh.at[pl.ds(pl.multiple_of(w * NR, 8), NR)],
                            out1_hbm.at[pl.ds(pl.multiple_of(w * NR, 8), NR)])

            @pl.when(w == 0)
            def _tail():
                pltpu.sync_copy(out1_sh.at[pl.ds(NT * NR, N - NT * NR)],
                                out1_hbm.at[pl.ds(NT * NR, N - NT * NR)])
            pltpu.sync_copy(cntst, nfo_hbm.at[pl.ds(pl.multiple_of(w * L, 8), L)])

    return sc_a


def _make_sc_c(N, E, D):
    EW = E // NT
    SCH = 64
    STG = 2048
    IDS_W = EW + STG

    scratch = [
        pltpu.VMEM_SHARED((NT * D,), jnp.float32),
        pltpu.VMEM_SHARED((NT * L,), jnp.float32),
        pltpu.VMEM_SHARED((NT * L,), jnp.float32),
        pltpu.HBM((NT * IDS_W,), jnp.float32),
        pltpu.VMEM((SCH,), jnp.int32),
        pltpu.VMEM((SCH,), jnp.int32),
        pltpu.VMEM((SCH,), jnp.int32),
        pltpu.VMEM((SCH,), jnp.float32),
        pltpu.VMEM((L,), jnp.int32),
        pltpu.VMEM((L,), jnp.int32),
        pltpu.VMEM((L,), jnp.float32),
        pltpu.VMEM((D, D), jnp.float32),
        pltpu.VMEM((D, D), jnp.float32),
        pltpu.VMEM((D,), jnp.float32),
        pltpu.VMEM((D,), jnp.float32),
        pltpu.VMEM((D,), jnp.float32),
        pltpu.VMEM((D,), jnp.float32),
        pltpu.VMEM((D,), jnp.float32),
        pltpu.VMEM((D,), jnp.float32),
        pltpu.VMEM((D,), jnp.float32),
        pltpu.VMEM((D,), jnp.float32),
        pltpu.VMEM((D,), jnp.float32),
        pltpu.VMEM((L, D), jnp.float32),
        pltpu.VMEM((NT * L,), jnp.float32),
        pltpu.VMEM((NT * D,), jnp.float32),
    ]

    @functools.partial(
        pl.kernel,
        out_type=jax.ShapeDtypeStruct((D,), jnp.float32),
        mesh=_mesh(),
        scratch_types=scratch,
        compiler_params=pltpu.CompilerParams(needs_layout_passes=False),
        interpret=_INTERPRET,
    )
    def sc_c(src_hbm, dst_hbm, nidx_hbm, nfo_hbm, ids_hbm, out1_hbm,
             c1_hbm, w2s_hbm, b2s_hbm, w2d_hbm, b2d_hbm, a2_hbm, c2_hbm,
             out_hbm,
             osh, d2sh, m2sh, ebuf_hb, idsc, sv, dv, echunk,
             nidxv, cntv, fst, w2sc, w2dc, c1c, b2sc, b2dc, a2c, c2c,
             hd2, h1n, qacc, outv, rstage, mall, qall):
        cid = lax.axis_index("c")
        w = lax.axis_index("s")

        def popcnt(m):
            return plsc.all_reduce_population_count(m)[0]

        def splat_at(ref, i):
            return plsc.load_gather(ref, [jnp.full((L,), i, jnp.int32)])

        @pl.when(cid == 0)
        def _main():
            iot = lax.iota(jnp.int32, L)
            zf = jnp.zeros((L,), jnp.float32)

            pltpu.sync_copy(nidx_hbm, nidxv)
            pltpu.sync_copy(nfo_hbm.at[pl.ds(pl.multiple_of(w * L, 8), L)], cntv)
            pltpu.sync_copy(a2_hbm, a2c)
            pltpu.sync_copy(c2_hbm, c2c)
            pltpu.sync_copy(c1_hbm, c1c)
            pltpu.sync_copy(b2s_hbm, b2sc)
            pltpu.sync_copy(b2d_hbm, b2dc)
            pltpu.sync_copy(w2s_hbm, w2sc)
            pltpu.sync_copy(w2d_hbm, w2dc)
            nvec = nidxv[:]
            nsc = nvec[0]
            n1 = cntv[:][0]

            pltpu.sync_copy(out1_hbm.at[nvec], rstage)

            def cphd(jj, _):
                h1n[pl.ds(jj * L, L)] = _gelu(rstage[0, pl.ds(jj * L, L)]
                                              + c1c[pl.ds(jj * L, L)])
                return 0
            lax.fori_loop(0, D // L, cphd, 0)

            def mvd(jj, _):
                acc = b2dc[pl.ds(jj * L, L)]

                def mvi(d, a):
                    return a + splat_at(h1n, d) * w2dc[d, pl.ds(jj * L, L)]
                acc = lax.fori_loop(0, D, mvi, acc)
                hd2[pl.ds(jj * L, L)] = acc
                return 0
            lax.fori_loop(0, D // L, mvd, 0)

            def zacc(jj, _):
                qacc[pl.ds(jj * L, L)] = zf
                return 0
            lax.fori_loop(0, D // L, zacc, 0)

            ntrip = (n1 + SCH - 1) // SCH

            def load_h1(ssc):
                pltpu.sync_copy(out1_hbm.at[jnp.full((L,), ssc, jnp.int32)], rstage)

                def g(jj, _):
                    h1n[pl.ds(jj * L, L)] = _gelu(rstage[0, pl.ds(jj * L, L)]
                                                  + c1c[pl.ds(jj * L, L)])
                    return 0
                lax.fori_loop(0, D // L, g, 0)

            def c1_outer(t, m2v):
                base = pl.multiple_of(t * SCH, 8)
                pltpu.sync_copy(ids_hbm.at[pl.ds(pl.multiple_of(w * IDS_W + base, 8), SCH)], idsc)

                def _cl(g, _0):
                    v = idsc[pl.ds(g * L, L)]
                    idsc[pl.ds(g * L, L)] = jnp.clip(v, 0, E - 1)
                    return 0
                lax.fori_loop(0, SCH // L, _cl, 0)
                pltpu.sync_copy(src_hbm.at[idsc], sv)
                pltpu.sync_copy(dst_hbm.at[idsc], dv)

                def init_e(j, _2):
                    echunk[pl.ds(j * L, L)] = jnp.full((L,), NEG, jnp.float32)
                    return 0
                lax.fori_loop(0, SCH // L, init_e, 0)

                def c1_edge(i, m2i):
                    dsc = splat_at(dv, i)[0]
                    is_l2 = (dsc == nsc) & ((base + i) < n1)

                    def do_edge(m2j):
                        ssc = splat_at(sv, i)[0]
                        load_h1(ssc)
                        e2acc = zf
                        for jj in range(D // L):
                            accj = b2sc[pl.ds(jj * L, L)]

                            def mv2(d, a):
                                return a + splat_at(h1n, d) * w2sc[d, pl.ds(jj * L, L)]
                            accj = lax.fori_loop(0, D, mv2, accj)
                            mj = accj + hd2[pl.ds(jj * L, L)]
                            e2acc = e2acc + _lrelu(mj) * a2c[pl.ds(jj * L, L)]
                        e2 = _vsum(e2acc)
                        plsc.store_scatter(echunk, [jnp.full((L,), i, jnp.int32)],
                                           jnp.full((L,), e2, jnp.float32),
                                           mask=iot == 0)
                        return jnp.maximum(m2j, jnp.full((L,), e2, jnp.float32))
                    return lax.cond(is_l2, do_edge, lambda m: m, m2i)
                m2v = lax.fori_loop(0, SCH, c1_edge, m2v)
                pltpu.sync_copy(echunk, ebuf_hb.at[pl.ds(pl.multiple_of(w * IDS_W + base, 8), SCH)])
                return m2v
            m2v = lax.fori_loop(0, ntrip, c1_outer,
                                jnp.full((L,), NEG, jnp.float32))
            fst[:] = m2v
            pltpu.sync_copy(fst, m2sh.at[pl.ds(pl.multiple_of(w * L, 8), L)])
            plsc.subcore_barrier()
            pltpu.sync_copy(m2sh, mall)
            m2g = jnp.full((L,), NEG, jnp.float32)
            for r in range(NT):
                m2g = jnp.maximum(m2g, mall[pl.ds(r * L, L)])

            def c2_outer(t, d2v):
                base = pl.multiple_of(t * SCH, 8)
                pltpu.sync_copy(ids_hbm.at[pl.ds(pl.multiple_of(w * IDS_W + base, 8), SCH)], idsc)

                def _cl(g, _0):
                    v = idsc[pl.ds(g * L, L)]
                    idsc[pl.ds(g * L, L)] = jnp.clip(v, 0, E - 1)
                    return 0
                lax.fori_loop(0, SCH // L, _cl, 0)
                pltpu.sync_copy(src_hbm.at[idsc], sv)
                pltpu.sync_copy(ebuf_hb.at[pl.ds(pl.multiple_of(w * IDS_W + base, 8), SCH)], echunk)

                def c2_edge(i, d2i):
                    e2sp = splat_at(echunk, i)
                    is_l2 = e2sp[0] > (0.5 * NEG)

                    def do_edge(d2j):
                        p2v = jnp.exp(e2sp - m2g)
                        ssc = splat_at(sv, i)[0]
                        load_h1(ssc)
                        for jj in range(D // L):
                            qacc[pl.ds(jj * L, L)] = (qacc[pl.ds(jj * L, L)]
                                                      + p2v * h1n[pl.ds(jj * L, L)])
                        return d2j + p2v
                    return lax.cond(is_l2, do_edge, lambda d: d, d2i)
                return lax.fori_loop(0, SCH, c2_edge, d2v)
            d2v = lax.fori_loop(0, ntrip, c2_outer, zf)

            pltpu.sync_copy(qacc, osh.at[pl.ds(pl.multiple_of(w * D, 8), D)])
            fst[:] = d2v
            pltpu.sync_copy(fst, d2sh.at[pl.ds(pl.multiple_of(w * L, 8), L)])
            plsc.subcore_barrier()

            @pl.when(w == 0)
            def _merge():
                pltpu.sync_copy(d2sh, mall)
                pltpu.sync_copy(osh, qall)
                d2g = zf
                for r in range(NT):
                    d2g = d2g + mall[pl.ds(r * L, L)]
                inv = 1.0 / (d2g + 1e-9)
                sal = d2g * inv

                def qsum(jj, _):
                    acc = zf
                    for r in range(NT):
                        acc = acc + qall[pl.ds(r * D + jj * L, L)]
                    qacc[pl.ds(jj * L, L)] = acc
                    return 0
                lax.fori_loop(0, D // L, qsum, 0)

                def mvout(jj, _):
                    acc = zf

                    def mvi(d, a):
                        return a + splat_at(qacc, d) * w2sc[d, pl.ds(jj * L, L)]
                    acc = lax.fori_loop(0, D, mvi, acc)
                    outv[pl.ds(jj * L, L)] = (acc * inv + sal * b2sc[pl.ds(jj * L, L)]
                                              + c2c[pl.ds(jj * L, L)])
                    return 0
                lax.fori_loop(0, D // L, mvout, 0)
                pltpu.sync_copy(outv, out_hbm)

    return sc_c


def kernel(x, edge_index, node_index, W0, W1s, b1s, W1d, b1d, a1, c1,
           W2s, b2s, W2d, b2d, a2, c2):
    N, _ = x.shape
    D = W0.shape[1]
    E = edge_index.shape[1]
    src = edge_index[0]
    dst = edge_index[1]
    nidx16 = jnp.full((L,), node_index, dtype=jnp.int32)
    hs1, hd1 = _tc_a(x, W0, W1s, b1s, W1d, b1d)
    out1, ids, nfo = _make_sc_a(N, E, D)(src, dst, nidx16, hs1, hd1, a1)
    return _make_sc_c(N, E, D)(src, dst, nidx16, nfo, ids, out1,
                               c1, W2s, b2s, W2d, b2d, a2, c2)


# speedup vs baseline: 1.0701x; 1.0701x over previous
import functools

import jax
import jax.numpy as jnp
from jax import lax
from jax.experimental import pallas as pl
from jax.experimental.pallas import tpu as pltpu
from jax.experimental.pallas import tpu_sc as plsc

L = 16
NT = 16
NEG = -1e30


def _tc_a(x, W0, W1s, b1s, W1d, b1d):
    N, D_IN = x.shape
    D = W0.shape[1]
    BLK = 1000

    def body(x_ref, w0_ref, w1s_ref, b1s_ref, w1d_ref, b1d_ref, hs_ref, hd_ref):
        h = jnp.dot(x_ref[...], w0_ref[...], preferred_element_type=jnp.float32)
        hs_ref[...] = jnp.dot(h, w1s_ref[...], preferred_element_type=jnp.float32) + b1s_ref[...]
        hd_ref[...] = jnp.dot(h, w1d_ref[...], preferred_element_type=jnp.float32) + b1d_ref[...]

    return pl.pallas_call(
        body,
        grid=(N // BLK,),
        in_specs=[
            pl.BlockSpec((BLK, D_IN), lambda i: (i, 0)),
            pl.BlockSpec((D_IN, D), lambda i: (0, 0)),
            pl.BlockSpec((D, D), lambda i: (0, 0)),
            pl.BlockSpec((1, D), lambda i: (0, 0)),
            pl.BlockSpec((D, D), lambda i: (0, 0)),
            pl.BlockSpec((1, D), lambda i: (0, 0)),
        ],
        out_specs=[
            pl.BlockSpec((BLK, D), lambda i: (i, 0)),
            pl.BlockSpec((BLK, D), lambda i: (i, 0)),
        ],
        out_shape=[
            jax.ShapeDtypeStruct((N, D), jnp.float32),
            jax.ShapeDtypeStruct((N, D), jnp.float32),
        ],
    )(x, W0, W1s, b1s.reshape(1, D), W1d, b1d.reshape(1, D))


def _tc_b(out1, c1, W2s, b2s, W2d, b2d):
    N, D = out1.shape
    BLK = 1000

    def body(o_ref, c1_ref, w2s_ref, b2s_ref, w2d_ref, b2d_ref, hs_ref, hd_ref):
        h1 = jax.nn.gelu(o_ref[...] + c1_ref[...])
        hs_ref[...] = jnp.dot(h1, w2s_ref[...], preferred_element_type=jnp.float32) + b2s_ref[...]
        hd_ref[...] = jnp.dot(h1, w2d_ref[...], preferred_element_type=jnp.float32) + b2d_ref[...]

    return pl.pallas_call(
        body,
        grid=(N // BLK,),
        in_specs=[
            pl.BlockSpec((BLK, D), lambda i: (i, 0)),
            pl.BlockSpec((1, D), lambda i: (0, 0)),
            pl.BlockSpec((D, D), lambda i: (0, 0)),
            pl.BlockSpec((1, D), lambda i: (0, 0)),
            pl.BlockSpec((D, D), lambda i: (0, 0)),
            pl.BlockSpec((1, D), lambda i: (0, 0)),
        ],
        out_specs=[
            pl.BlockSpec((BLK, D), lambda i: (i, 0)),
            pl.BlockSpec((BLK, D), lambda i: (i, 0)),
        ],
        out_shape=[
            jax.ShapeDtypeStruct((N, D), jnp.float32),
            jax.ShapeDtypeStruct((N, D), jnp.float32),
        ],
    )(out1, c1.reshape(1, D), W2s, b2s.reshape(1, D), W2d, b2d.reshape(1, D))


def _lrelu(t):
    return jnp.maximum(t, 0.2 * t)


def _vsum(v):
    s = v[0]
    for k in range(1, L):
        s = s + v[k]
    return s


def _mesh():
    return plsc.VectorSubcoreMesh(core_axis_name="c", subcore_axis_name="s",
                                  num_cores=2, num_subcores=16)


def _make_sc_a(N, E, D):
    UM = 12288
    DUMP = N
    EW = E // NT
    CH = 2000
    SCH = 64
    STG = 2048
    IDS_W = EW + STG
    RPT = (N + L) // NT

    scratch = [
        pltpu.VMEM_SHARED((N + L, D), jnp.float32),
        pltpu.VMEM_SHARED((UM,), jnp.int32),
        pltpu.VMEM_SHARED((UM,), jnp.float32),
        pltpu.VMEM_SHARED((NT * L,), jnp.float32),
        pltpu.HBM((NT * IDS_W,), jnp.float32),
        pltpu.VMEM((CH,), jnp.int32),
        pltpu.VMEM((CH,), jnp.int32),
        pltpu.VMEM((10240,), jnp.int32),
        pltpu.VMEM((STG + L,), jnp.int32),
        pltpu.VMEM((L,), jnp.int32),
        pltpu.VMEM((L,), jnp.int32),
        pltpu.VMEM((SCH,), jnp.int32),
        pltpu.VMEM((SCH,), jnp.int32),
        pltpu.VMEM((SCH,), jnp.int32),
        pltpu.VMEM((SCH,), jnp.int32),
        pltpu.VMEM((SCH,), jnp.float32),
        pltpu.VMEM((SCH, D), jnp.float32),
        pltpu.VMEM((SCH, D), jnp.float32),
        pltpu.VMEM((L, D), jnp.float32),
        pltpu.VMEM((SCH,), jnp.float32),
        pltpu.VMEM((SCH,), jnp.float32),
        pltpu.VMEM((L,), jnp.int32),
        pltpu.VMEM((D,), jnp.float32),
        pltpu.VMEM((L,), jnp.int32),
        pltpu.VMEM((L,), jnp.float32),
        pltpu.VMEM((STG,), jnp.float32),
        pltpu.VMEM((NT * L,), jnp.float32),
        pltpu.VMEM((CH,), jnp.int32),
        pltpu.VMEM((CH // L + 3,), jnp.int32),
        pltpu.VMEM((CH,), jnp.int32),
        pltpu.SemaphoreType.DMA,
        pltpu.SemaphoreType.DMA,
    ]

    @functools.partial(
        pl.kernel,
        out_type=(
            jax.ShapeDtypeStruct((N, D), jnp.float32),
            jax.ShapeDtypeStruct((NT * IDS_W,), jnp.int32),
            jax.ShapeDtypeStruct((NT * L,), jnp.int32),
        ),
        mesh=_mesh(),
        scratch_types=scratch,
        compiler_params=pltpu.CompilerParams(needs_layout_passes=False),
    )
    def sc_a(src_hbm, dst_hbm, nidx_hbm, hs1_hbm, hd1_hbm, a1_hbm,
             out1_hbm, ids_hbm, nfo_hbm,
             out1_sh, umask_sh, denom_sh, m1_sh, ebuf_hb,
             dstc, srcc, umask, stage, stg16, onesv,
             idsc, sv, dv, dvw, denc, rows_s, rows_d, rows16,
             echunk, alphac, nidxv, a1c, cntst, m1st, zstagef, m1all, cbuf, cntb,
             dstc2, semA, semB):
        cid = lax.axis_index("c")
        w = lax.axis_index("s")

        def popcnt(m):
            return plsc.all_reduce_population_count(m)[0]

        @pl.when(cid == 0)
        def _main():
            iot = lax.iota(jnp.int32, L)
            zi = jnp.zeros((L,), jnp.int32)
            zf = jnp.zeros((L,), jnp.float32)
            onei = jnp.ones((L,), jnp.int32)
            ebase = w * EW

            pltpu.sync_copy(nidx_hbm, nidxv)
            pltpu.sync_copy(a1_hbm, a1c)
            nvec = nidxv[:]
            onesv[:] = onei

            def stg_body(j, _):
                stage[pl.ds(j * L, L)] = zi
                return 0
            lax.fori_loop(0, (STG + L) // L, stg_body, 0)

            def stgf_body(j, _):
                zstagef[pl.ds(j * L, L)] = zf
                return 0
            lax.fori_loop(0, STG // L, stgf_body, 0)

            def zrow_body(j, _):
                def zcol(k, _2):
                    rows16[j, pl.ds(k * L, L)] = zf
                    return 0
                lax.fori_loop(0, D // L, zcol, 0)
                return 0
            lax.fori_loop(0, L, zrow_body, 0)

            @pl.when(w == 0)
            def _zero_shared():
                def zs(j, _):
                    pltpu.sync_copy(stage.at[pl.ds(0, STG)],
                                    umask_sh.at[pl.ds(j * STG, STG)])
                    return 0
                lax.fori_loop(0, UM // STG, zs, 0)

                def zd(j, _):
                    pltpu.sync_copy(zstagef.at[pl.ds(0, STG)],
                                    denom_sh.at[pl.ds(j * STG, STG)])
                    return 0
                lax.fori_loop(0, UM // STG, zd, 0)

            @pl.when(w == 0)
            def _zero_dump():
                pltpu.sync_copy(rows16, out1_sh.at[pl.ds(N, L)])

            plsc.subcore_barrier()

            fvec = jnp.zeros((L,), jnp.bool_)
            NCH = EW // CH

            def p1_proc(t, dref):
                off = pl.multiple_of(ebase + t * CH, 8)

                def p1_scan(j, macc):
                    return macc | (dref[pl.ds(j * L, L)] == nvec)
                macc = lax.fori_loop(0, CH // L, p1_scan, fvec)

                @pl.when(popcnt(macc) > 0)
                def _rescan():
                    pltpu.sync_copy(src_hbm.at[pl.ds(off, CH)], srcc)

                    def p1_inner(j, _2):
                        d16 = dref[pl.ds(j * L, L)]
                        m = d16 == nvec

                        @pl.when(popcnt(m) > 0)
                        def _mark():
                            s16 = srcc[pl.ds(j * L, L)]
                            idxw = jnp.where(m, s16, DUMP)
                            pltpu.sync_copy(onesv, umask_sh.at[idxw])
                        return 0
                    lax.fori_loop(0, CH // L, p1_inner, 0)

            def p1_start(t, dref, sem):
                off = pl.multiple_of(ebase + (t % NCH) * CH, 8)
                return pltpu.async_copy(dst_hbm.at[pl.ds(off, CH)], dref, sem)

            p1_start(0, dstc, semA)

            def p1_pair(p, _):
                pltpu.make_async_copy(dst_hbm.at[pl.ds(0, CH)], dstc, semA).wait()
                p1_start(2 * p + 1, dstc2, semB)
                p1_proc(2 * p, dstc)
                pltpu.make_async_copy(dst_hbm.at[pl.ds(0, CH)], dstc2, semB).wait()
                p1_start(2 * p + 2, dstc, semA)
                p1_proc(2 * p + 1, dstc2)
                return 0
            lax.fori_loop(0, NCH // 2, p1_pair, 0)
            pltpu.make_async_copy(dst_hbm.at[pl.ds(0, CH)], dstc, semA).wait()

            plsc.subcore_barrier()

            pltpu.sync_copy(umask_sh.at[pl.ds(0, 10240)], umask)
            plsc.store_scatter(umask, [nvec], onei)

            UG = 10240 // (L * NT)
            def zu_body(g, _):
                gg = w * UG + g
                um16 = umask[pl.ds(gg * L, L)]
                m = um16 > 0

                @pl.when(popcnt(m) > 0)
                def _z():
                    idxw = jnp.where(m, gg * L + iot, DUMP)
                    pltpu.sync_copy(rows16, out1_sh.at[idxw])
                return 0
            lax.fori_loop(0, UG, zu_body, 0)
            plsc.subcore_barrier()

            def p2_outer(t, carry, dref):
                def p2_stageA(j, _2):
                    d16 = dref[pl.ds(j * L, L)]
                    um = plsc.load_gather(umask, [d16])
                    m = um > 0
                    eid = ebase + t * CH + j * L + iot
                    plsc.store_compressed(cbuf.at[pl.ds(j * L, L)], eid, mask=m)
                    cs = plsc.all_reduce_population_count(m)
                    plsc.store_scatter(cntb, [jnp.full((L,), j, jnp.int32)],
                                       cs, mask=iot == 0)
                    return 0
                lax.fori_loop(0, CH // L, p2_stageA, 0)

                def p2_stageB(j, carry2):
                    cnt2, off2, fb2 = carry2
                    c = plsc.load_gather(cntb, [jnp.full((L,), j, jnp.int32)])[0]
                    stage[pl.ds(off2, L)] = cbuf[pl.ds(j * L, L)]
                    off3 = off2 + c

                    @pl.when(off3 >= STG)
                    def _flush():
                        pltpu.sync_copy(
                            stage.at[pl.ds(0, STG)],
                            ids_hbm.at[pl.ds(pl.multiple_of(w * IDS_W + fb2, 8), STG)])
                        stage[pl.ds(0, L)] = stage[pl.ds(STG, L)]

                    wrapped = off3 >= STG
                    off4 = jnp.where(wrapped, off3 - STG, off3)
                    fb3 = jnp.where(wrapped, fb2 + STG, fb2)
                    return (cnt2 + c, off4, fb3)
                return lax.fori_loop(0, CH // L, p2_stageB, carry)

            p1_start(0, dstc, semA)

            def p2_pair(p, carry):
                pltpu.make_async_copy(dst_hbm.at[pl.ds(0, CH)], dstc, semA).wait()
                p1_start(2 * p + 1, dstc2, semB)
                carry = p2_outer(2 * p, carry, dstc)
                pltpu.make_async_copy(dst_hbm.at[pl.ds(0, CH)], dstc2, semB).wait()
                p1_start(2 * p + 2, dstc, semA)
                carry = p2_outer(2 * p + 1, carry, dstc2)
                return carry
            n1, offr, fbr = lax.fori_loop(0, NCH // 2, p2_pair,
                                          (jnp.int32(0), jnp.int32(0), jnp.int32(0)))
            pltpu.make_async_copy(dst_hbm.at[pl.ds(0, CH)], dstc, semA).wait()
            pltpu.sync_copy(stage.at[pl.ds(0, STG)],
                            ids_hbm.at[pl.ds(pl.multiple_of(w * IDS_W + fbr, 8), STG)])

            cntst[:] = jnp.full((L,), n1, jnp.int32)

            ntrip = (n1 + SCH - 1) // SCH

            def p3_outer(t, m1v):
                base = pl.multiple_of(t * SCH, 8)
                pltpu.sync_copy(ids_hbm.at[pl.ds(pl.multiple_of(w * IDS_W + base, 8), SCH)], idsc)

                def _cl(g, _0):
                    v = idsc[pl.ds(g * L, L)]
                    idsc[pl.ds(g * L, L)] = jnp.clip(v, 0, E - 1)
                    return 0
                lax.fori_loop(0, SCH // L, _cl, 0)
                pltpu.sync_copy(src_hbm.at[idsc], sv)
                pltpu.sync_copy(dst_hbm.at[idsc], dv)
                pltpu.sync_copy(hs1_hbm.at[sv], rows_s)
                pltpu.sync_copy(hd1_hbm.at[dv], rows_d)

                def p3_edge(i, m1i):
                    acc = zf
                    for jj in range(D // L):
                        t1 = rows_s[i, pl.ds(jj * L, L)] + rows_d[i, pl.ds(jj * L, L)]
                        acc = acc + _lrelu(t1) * a1c[pl.ds(jj * L, L)]
                    e = _vsum(acc)
                    plsc.store_scatter(echunk, [jnp.full((L,), i, jnp.int32)],
                                       jnp.full((L,), e, jnp.float32), mask=iot == 0)
                    valid = (base + i) < n1
                    ev = jnp.full((L,), jnp.where(valid, e, NEG), jnp.float32)
                    return jnp.maximum(m1i, ev)
                m1v = lax.fori_loop(0, SCH, p3_edge, m1v)
                pltpu.sync_copy(echunk, ebuf_hb.at[pl.ds(pl.multiple_of(w * IDS_W + base, 8), SCH)])
                return m1v
            m1v = lax.fori_loop(0, ntrip, p3_outer,
                                jnp.full((L,), NEG, jnp.float32))
            m1st[:] = m1v
            pltpu.sync_copy(m1st, m1_sh.at[pl.ds(pl.multiple_of(w * L, 8), L)])
            plsc.subcore_barrier()
            pltpu.sync_copy(m1_sh, m1all)
            m1g = jnp.full((L,), NEG, jnp.float32)
            for r in range(NT):
                m1g = jnp.maximum(m1g, m1all[pl.ds(r * L, L)])

            def p4_outer(t, _):
                base = pl.multiple_of(t * SCH, 8)
                pltpu.sync_copy(ids_hbm.at[pl.ds(pl.multiple_of(w * IDS_W + base, 8), SCH)], idsc)

                def _cl(g, _0):
                    v = idsc[pl.ds(g * L, L)]
                    idsc[pl.ds(g * L, L)] = jnp.clip(v, 0, E - 1)
                    return 0
                lax.fori_loop(0, SCH // L, _cl, 0)
                pltpu.sync_copy(dst_hbm.at[idsc], dv)
                pltpu.sync_copy(ebuf_hb.at[pl.ds(pl.multiple_of(w * IDS_W + base, 8), SCH)], echunk)

                def p4_grp(j, _2):
                    e16 = echunk[pl.ds(j * L, L)]
                    d16 = dv[pl.ds(j * L, L)]
                    vmask = (base + j * L + iot) < n1
                    p = jnp.where(vmask, jnp.exp(e16 - m1g), 0.0)
                    echunk[pl.ds(j * L, L)] = p
                    dvw[pl.ds(j * L, L)] = jnp.where(vmask, d16, DUMP)
                    return 0
                lax.fori_loop(0, SCH // L, p4_grp, 0)
                pltpu.sync_copy(echunk, ebuf_hb.at[pl.ds(pl.multiple_of(w * IDS_W + base, 8), SCH)])
                pltpu.sync_copy(echunk, denom_sh.at[dvw], add=True)
                return 0
            lax.fori_loop(0, ntrip, p4_outer, 0)
            plsc.subcore_barrier()

            def p5_outer(t, _):
                base = pl.multiple_of(t * SCH, 8)
                pltpu.sync_copy(ids_hbm.at[pl.ds(pl.multiple_of(w * IDS_W + base, 8), SCH)], idsc)

                def _cl(g, _0):
                    v = idsc[pl.ds(g * L, L)]
                    idsc[pl.ds(g * L, L)] = jnp.clip(v, 0, E - 1)
                    return 0
                lax.fori_loop(0, SCH // L, _cl, 0)
                pltpu.sync_copy(src_hbm.at[idsc], sv)
                pltpu.sync_copy(dst_hbm.at[idsc], dv)
                pltpu.sync_copy(ebuf_hb.at[pl.ds(pl.multiple_of(w * IDS_W + base, 8), SCH)], echunk)
                pltpu.sync_copy(hs1_hbm.at[sv], rows_s)

                def p5_grp(j, _2):
                    d16 = dv[pl.ds(j * L, L)]
                    vmask = (base + j * L + iot) < n1
                    dvw[pl.ds(j * L, L)] = jnp.where(vmask, d16, DUMP)
                    return 0
                lax.fori_loop(0, SCH // L, p5_grp, 0)
                pltpu.sync_copy(denom_sh.at[dvw], denc)

                def p5_alpha(j, _2):
                    p16 = echunk[pl.ds(j * L, L)]
                    den = denc[pl.ds(j * L, L)]
                    vmask = (base + j * L + iot) < n1
                    al = jnp.where(vmask, p16 / (den + 1e-9), 0.0)
                    alphac[pl.ds(j * L, L)] = al
                    return 0
                lax.fori_loop(0, SCH // L, p5_alpha, 0)

                def p5_edge(i, _2):
                    asp = plsc.load_gather(alphac, [jnp.full((L,), i, jnp.int32)])
                    for jj in range(D // L):
                        rows_d[i, pl.ds(jj * L, L)] = rows_s[i, pl.ds(jj * L, L)] * asp
                    return 0
                lax.fori_loop(0, SCH, p5_edge, 0)
                pltpu.sync_copy(rows_d, out1_sh.at[dvw], add=True)
                return 0
            lax.fori_loop(0, ntrip, p5_outer, 0)
            plsc.subcore_barrier()

            NR = (N // NT) // 8 * 8
            pltpu.sync_copy(out1_sh.at[pl.ds(pl.multiple_of(w * NR, 8), NR)],
                            out1_hbm.at[pl.ds(pl.multiple_of(w * NR, 8), NR)])

            @pl.when(w == 0)
            def _tail():
                pltpu.sync_copy(out1_sh.at[pl.ds(NT * NR, N - NT * NR)],
                                out1_hbm.at[pl.ds(NT * NR, N - NT * NR)])
            pltpu.sync_copy(cntst, nfo_hbm.at[pl.ds(pl.multiple_of(w * L, 8), L)])

    return sc_a


def _make_sc_c(N, E, D):
    EW = E // NT
    SCH = 64
    STG = 2048
    IDS_W = EW + STG

    scratch = [
        pltpu.VMEM_SHARED((NT * D,), jnp.float32),
        pltpu.VMEM_SHARED((NT * L,), jnp.float32),
        pltpu.VMEM_SHARED((NT * L,), jnp.float32),
        pltpu.HBM((NT * IDS_W,), jnp.float32),
        pltpu.VMEM((SCH,), jnp.int32),
        pltpu.VMEM((SCH,), jnp.int32),
        pltpu.VMEM((SCH,), jnp.int32),
        pltpu.VMEM((SCH, D), jnp.float32),
        pltpu.VMEM((SCH,), jnp.float32),
        pltpu.VMEM((L,), jnp.int32),
        pltpu.VMEM((L,), jnp.int32),
        pltpu.VMEM((L,), jnp.float32),
        pltpu.VMEM((D,), jnp.float32),
        pltpu.VMEM((D,), jnp.float32),
        pltpu.VMEM((D,), jnp.float32),
        pltpu.VMEM((D,), jnp.float32),
        pltpu.VMEM((L, D), jnp.float32),
        pltpu.VMEM((NT * L,), jnp.float32),
        pltpu.VMEM((NT * D,), jnp.float32),
    ]

    @functools.partial(
        pl.kernel,
        out_type=jax.ShapeDtypeStruct((D,), jnp.float32),
        mesh=_mesh(),
        scratch_types=scratch,
        compiler_params=pltpu.CompilerParams(needs_layout_passes=False),
    )
    def sc_c(src_hbm, dst_hbm, nidx_hbm, nfo_hbm, ids_hbm, hs2_hbm, hd2_hbm,
             a2_hbm, c2_hbm, out_hbm,
             osh, d2sh, m2sh, ebuf_hb, idsc, sv, dv, rows_s, echunk,
             nidxv, cntv, fst, a2c, c2c, hd2, outv, rstage, mall, oall):
        cid = lax.axis_index("c")
        w = lax.axis_index("s")

        @pl.when(cid == 0)
        def _main():
            iot = lax.iota(jnp.int32, L)
            zf = jnp.zeros((L,), jnp.float32)

            pltpu.sync_copy(nidx_hbm, nidxv)
            pltpu.sync_copy(nfo_hbm.at[pl.ds(pl.multiple_of(w * L, 8), L)], cntv)
            pltpu.sync_copy(a2_hbm, a2c)
            pltpu.sync_copy(c2_hbm, c2c)
            nvec = nidxv[:]
            nsc = nvec[0]
            n1 = cntv[:][0]

            pltpu.sync_copy(hd2_hbm.at[nvec], rstage)

            def cphd(jj, _):
                hd2[pl.ds(jj * L, L)] = rstage[0, pl.ds(jj * L, L)]
                return 0
            lax.fori_loop(0, D // L, cphd, 0)

            def zacc(jj, _):
                outv[pl.ds(jj * L, L)] = zf
                return 0
            lax.fori_loop(0, D // L, zacc, 0)

            ntrip = (n1 + SCH - 1) // SCH

            def c1_outer(t, m2v):
                base = pl.multiple_of(t * SCH, 8)
                pltpu.sync_copy(ids_hbm.at[pl.ds(pl.multiple_of(w * IDS_W + base, 8), SCH)], idsc)

                def _cl(g, _0):
                    v = idsc[pl.ds(g * L, L)]
                    idsc[pl.ds(g * L, L)] = jnp.clip(v, 0, E - 1)
                    return 0
                lax.fori_loop(0, SCH // L, _cl, 0)
                pltpu.sync_copy(src_hbm.at[idsc], sv)
                pltpu.sync_copy(dst_hbm.at[idsc], dv)
                pltpu.sync_copy(hs2_hbm.at[sv], rows_s)

                def c1_edge(i, m2i):
                    acc = zf
                    for jj in range(D // L):
                        t1 = rows_s[i, pl.ds(jj * L, L)] + hd2[pl.ds(jj * L, L)]
                        acc = acc + _lrelu(t1) * a2c[pl.ds(jj * L, L)]
                    e2 = _vsum(acc)
                    dsp = plsc.load_gather(dv, [jnp.full((L,), i, jnp.int32)])
                    is_l2 = (dsp[0] == nsc) & ((base + i) < n1)
                    e2w = jnp.where(is_l2, e2, NEG)
                    plsc.store_scatter(echunk, [jnp.full((L,), i, jnp.int32)],
                                       jnp.full((L,), e2w, jnp.float32),
                                       mask=iot == 0)
                    return jnp.maximum(m2i, jnp.full((L,), e2w, jnp.float32))
                m2v = lax.fori_loop(0, SCH, c1_edge, m2v)
                pltpu.sync_copy(echunk, ebuf_hb.at[pl.ds(pl.multiple_of(w * IDS_W + base, 8), SCH)])
                return m2v
            m2v = lax.fori_loop(0, ntrip, c1_outer,
                                jnp.full((L,), NEG, jnp.float32))
            fst[:] = m2v
            pltpu.sync_copy(fst, m2sh.at[pl.ds(pl.multiple_of(w * L, 8), L)])
            plsc.subcore_barrier()
            pltpu.sync_copy(m2sh, mall)
            m2g = jnp.full((L,), NEG, jnp.float32)
            for r in range(NT):
                m2g = jnp.maximum(m2g, mall[pl.ds(r * L, L)])

            def c2_outer(t, d2v):
                base = pl.multiple_of(t * SCH, 8)
                pltpu.sync_copy(ids_hbm.at[pl.ds(pl.multiple_of(w * IDS_W + base, 8), SCH)], idsc)

                def _cl(g, _0):
                    v = idsc[pl.ds(g * L, L)]
                    idsc[pl.ds(g * L, L)] = jnp.clip(v, 0, E - 1)
                    return 0
                lax.fori_loop(0, SCH // L, _cl, 0)
                pltpu.sync_copy(src_hbm.at[idsc], sv)
                pltpu.sync_copy(ebuf_hb.at[pl.ds(pl.multiple_of(w * IDS_W + base, 8), SCH)], echunk)
                pltpu.sync_copy(hs2_hbm.at[sv], rows_s)

                def c2_edge(i, d2i):
                    e2sp = plsc.load_gather(echunk, [jnp.full((L,), i, jnp.int32)])
                    is_l2 = e2sp[0] > (0.5 * NEG)

                    def do_edge(d2j):
                        p2v = jnp.exp(e2sp - m2g)
                        for jj in range(D // L):
                            outv[pl.ds(jj * L, L)] = (outv[pl.ds(jj * L, L)]
                                                      + p2v * rows_s[i, pl.ds(jj * L, L)])
                        return d2j + p2v
                    return lax.cond(is_l2, do_edge, lambda d: d, d2i)
                return lax.fori_loop(0, SCH, c2_edge, d2v)
            d2v = lax.fori_loop(0, ntrip, c2_outer, zf)

            pltpu.sync_copy(outv, osh.at[pl.ds(pl.multiple_of(w * D, 8), D)])
            fst[:] = d2v
            pltpu.sync_copy(fst, d2sh.at[pl.ds(pl.multiple_of(w * L, 8), L)])
            plsc.subcore_barrier()

            @pl.when(w == 0)
            def _merge():
                pltpu.sync_copy(d2sh, mall)
                pltpu.sync_copy(osh, oall)
                d2g = zf
                for r in range(NT):
                    d2g = d2g + mall[pl.ds(r * L, L)]
                inv = 1.0 / (d2g + 1e-9)

                def fin(jj, _):
                    acc = zf
                    for r in range(NT):
                        acc = acc + oall[pl.ds(r * D + jj * L, L)]
                    outv[pl.ds(jj * L, L)] = acc * inv + c2c[pl.ds(jj * L, L)]
                    return 0
                lax.fori_loop(0, D // L, fin, 0)
                pltpu.sync_copy(outv, out_hbm)

    return sc_c


def kernel(x, edge_index, node_index, W0, W1s, b1s, W1d, b1d, a1, c1,
           W2s, b2s, W2d, b2d, a2, c2):
    N, _ = x.shape
    D = W0.shape[1]
    E = edge_index.shape[1]
    src = edge_index[0]
    dst = edge_index[1]
    nidx16 = jnp.full((L,), node_index, dtype=jnp.int32)
    hs1, hd1 = _tc_a(x, W0, W1s, b1s, W1d, b1d)
    out1, ids, nfo = _make_sc_a(N, E, D)(src, dst, nidx16, hs1, hd1, a1)
    hs2, hd2 = _tc_b(out1, c1, W2s, b2s, W2d, b2d)
    return _make_sc_c(N, E, D)(src, dst, nidx16, nfo, ids, hs2, hd2, a2, c2)
